# baseline (speedup 1.0000x reference)
import functools

import jax
import jax.numpy as jnp
from jax import lax
from jax.experimental import pallas as pl
from jax.experimental.pallas import tpu as pltpu
from jax.experimental.pallas import tpu_sc as plsc

VOCAB, EMBED, HIDDEN, NGRAM, BATCH = 100000, 64, 512, 20, 1024
EPAD = 128
NTOK = BATCH * NGRAM
NW = 32
TOK_PER_W = NTOK // NW
CHUNK = 128
NCHUNK = TOK_PER_W // CHUNK

VB = 2048


def _sc_gather(table_pad, idx):
    mesh = plsc.VectorSubcoreMesh(core_axis_name="c", subcore_axis_name="s")

    @functools.partial(
        pl.kernel,
        mesh=mesh,
        out_type=jax.ShapeDtypeStruct((NTOK, EPAD), jnp.float32),
        scratch_types=[
            pltpu.VMEM((TOK_PER_W,), jnp.int32),
            pltpu.VMEM((TOK_PER_W, EPAD), jnp.float32),
            pltpu.SemaphoreType.DMA,
        ],
    )
    def gather_kernel(table_hbm, idx_hbm, out_hbm, idx_v, rows_v, sem):
        wid = lax.axis_index("s") * 2 + lax.axis_index("c")
        base = wid * TOK_PER_W
        pltpu.sync_copy(idx_hbm.at[pl.ds(base, TOK_PER_W)], idx_v)
        copies = []
        for j in range(NCHUNK):
            copies.append(
                pltpu.async_copy(
                    table_hbm.at[idx_v.at[pl.ds(j * CHUNK, CHUNK)]],
                    rows_v.at[pl.ds(j * CHUNK, CHUNK)],
                    sem,
                )
            )
        for c in copies:
            c.wait()
        pltpu.sync_copy(rows_v, out_hbm.at[pl.ds(base, TOK_PER_W)])

    return gather_kernel(table_pad, idx)


def _fc1_body(emb_ref, w1_ref, b1_ref, h_ref):
    acc = jnp.zeros((BATCH, HIDDEN), jnp.float32)
    for t in range(NGRAM):
        e_t = emb_ref[pl.ds(t * BATCH, BATCH), :]
        w_t = w1_ref[:, pl.ds(t * EPAD, EPAD)]
        acc += lax.dot_general(
            e_t, w_t, (((1,), (1,)), ((), ())),
            preferred_element_type=jnp.float32,
        )
    h_ref[...] = jnp.maximum(acc + b1_ref[...], 0.0).astype(jnp.bfloat16)


def _fc1(emb, W1p, b1):
    return pl.pallas_call(
        _fc1_body,
        out_shape=jax.ShapeDtypeStruct((BATCH, HIDDEN), jnp.bfloat16),
    )(emb, W1p, b1.reshape(1, HIDDEN))


def _fc2_body(h_ref, w2_ref, b2_ref, out_ref):
    acc = lax.dot_general(
        w2_ref[...].astype(jnp.bfloat16), h_ref[...], (((1,), (1,)), ((), ())),
        preferred_element_type=jnp.float32,
    )
    out_ref[...] = acc + b2_ref[...].reshape(VB, 1)


def _fc2(h, W2, b2):
    nb = pl.cdiv(VOCAB, VB)
    return pl.pallas_call(
        _fc2_body,
        grid=(nb,),
        in_specs=[
            pl.BlockSpec((BATCH, HIDDEN), lambda i: (0, 0)),
            pl.BlockSpec((VB, HIDDEN), lambda i: (i, 0)),
            pl.BlockSpec((VB,), lambda i: (i,)),
        ],
        out_specs=pl.BlockSpec((VB, BATCH), lambda i: (i, 0)),
        out_shape=jax.ShapeDtypeStruct((VOCAB, BATCH), jnp.float32),
    )(h, W2, b2)


def _mlp(emb, W1p, b1, W2, b2):
    h = _fc1(emb, W1p, b1)
    return _fc2(h, W2, b2).T


def kernel(x, table, W1, b1, W2, b2):
    table_pad = jnp.zeros((VOCAB, EPAD), jnp.float32).at[:, :EMBED].set(table)
    idx = x.T.reshape(NTOK)
    emb = _sc_gather(table_pad, idx)
    W1p = jnp.pad(W1.reshape(HIDDEN, NGRAM, EMBED),
                  ((0, 0), (0, 0), (0, EPAD - EMBED))).reshape(HIDDEN, -1)
    return _mlp(emb, W1p, b1, W2, b2)


# speedup vs baseline: 1.1229x; 1.1229x over previous
import functools

import jax
import jax.numpy as jnp
from jax import lax
from jax.experimental import pallas as pl
from jax.experimental.pallas import tpu as pltpu
from jax.experimental.pallas import tpu_sc as plsc

VOCAB, EMBED, HIDDEN, NGRAM, BATCH = 100000, 64, 512, 20, 1024
NTOK = BATCH * NGRAM
NW = 32
TOK_PER_W = NTOK // NW
CHUNK = 128
NCHUNK = TOK_PER_W // CHUNK
NPAIR = NGRAM // 2
PW = 2 * EMBED

VB = 2048


def _sc_gather(table, idx):
    mesh = plsc.VectorSubcoreMesh(core_axis_name="c", subcore_axis_name="s")

    @functools.partial(
        pl.kernel,
        mesh=mesh,
        out_type=jax.ShapeDtypeStruct((NW, TOK_PER_W, EMBED), jnp.float32),
        scratch_types=[
            pltpu.VMEM((TOK_PER_W,), jnp.int32),
            pltpu.VMEM((TOK_PER_W, EMBED), jnp.float32),
            pltpu.SemaphoreType.DMA,
        ],
        compiler_params=pltpu.CompilerParams(use_tc_tiling_on_sc=False),
    )
    def gather_kernel(table_hbm, idx_hbm, out_hbm, idx_v, rows_v, sem):
        wid = lax.axis_index("s") * 2 + lax.axis_index("c")
        base = wid * TOK_PER_W
        pltpu.sync_copy(idx_hbm.at[pl.ds(base, TOK_PER_W)], idx_v)
        copies = []
        for j in range(NCHUNK):
            copies.append(
                pltpu.async_copy(
                    table_hbm.at[idx_v.at[pl.ds(j * CHUNK, CHUNK)]],
                    rows_v.at[pl.ds(j * CHUNK, CHUNK)],
                    sem,
                )
            )
        for c in copies:
            c.wait()
        pltpu.sync_copy(rows_v, out_hbm.at[wid])

    return gather_kernel(table, idx)


def _fc1_body(emb_ref, w1_ref, b1_ref, h_ref):
    acc = jnp.zeros((BATCH, HIDDEN), jnp.float32)
    for s in range(NPAIR):
        e_s = emb_ref[pl.ds(s * BATCH, BATCH), :]
        w_s = w1_ref[:, pl.ds(s * PW, PW)]
        acc += lax.dot_general(
            e_s, w_s, (((1,), (1,)), ((), ())),
            preferred_element_type=jnp.float32,
        )
    h_ref[...] = jnp.maximum(acc + b1_ref[...], 0.0).astype(jnp.bfloat16)


def _fc1(emb2, W1, b1):
    return pl.pallas_call(
        _fc1_body,
        out_shape=jax.ShapeDtypeStruct((BATCH, HIDDEN), jnp.bfloat16),
    )(emb2, W1, b1.reshape(1, HIDDEN))


def _fc2_body(h_ref, w2_ref, b2_ref, out_ref):
    acc = lax.dot_general(
        w2_ref[...].astype(jnp.bfloat16), h_ref[...], (((1,), (1,)), ((), ())),
        preferred_element_type=jnp.float32,
    )
    out_ref[...] = acc + b2_ref[...].reshape(VB, 1)


def _fc2(h, W2, b2):
    nb = pl.cdiv(VOCAB, VB)
    return pl.pallas_call(
        _fc2_body,
        grid=(nb,),
        in_specs=[
            pl.BlockSpec((BATCH, HIDDEN), lambda i: (0, 0)),
            pl.BlockSpec((VB, HIDDEN), lambda i: (i, 0)),
            pl.BlockSpec((VB,), lambda i: (i,)),
        ],
        out_specs=pl.BlockSpec((VB, BATCH), lambda i: (i, 0)),
        out_shape=jax.ShapeDtypeStruct((VOCAB, BATCH), jnp.float32),
    )(h, W2, b2)


def kernel(x, table, W1, b1, W2, b2):
    xp = x.reshape(BATCH, NPAIR, 2).transpose(1, 0, 2).reshape(NTOK)
    emb3 = _sc_gather(table, xp)
    emb2 = emb3.reshape(NPAIR * BATCH, PW)
    h = _fc1(emb2, W1, b1)
    return _fc2(h, W2, b2).T


# speedup vs baseline: 1.1407x; 1.0159x over previous
import functools

import jax
import jax.numpy as jnp
from jax import lax
from jax.experimental import pallas as pl
from jax.experimental.pallas import tpu as pltpu
from jax.experimental.pallas import tpu_sc as plsc

VOCAB, EMBED, HIDDEN, NGRAM, BATCH = 100000, 64, 512, 20, 1024
NTOK = BATCH * NGRAM
NW = 32
TOK_PER_W = NTOK // NW
CHUNK = 128
NCHUNK = TOK_PER_W // CHUNK
NPAIR = NGRAM // 2
PW = 2 * EMBED

VB = 4096


def _sc_gather(table, idx):
    mesh = plsc.VectorSubcoreMesh(core_axis_name="c", subcore_axis_name="s")

    @functools.partial(
        pl.kernel,
        mesh=mesh,
        out_type=jax.ShapeDtypeStruct((NW, TOK_PER_W, EMBED), jnp.float32),
        scratch_types=[
            pltpu.VMEM((TOK_PER_W,), jnp.int32),
            pltpu.VMEM((TOK_PER_W, EMBED), jnp.float32),
            pltpu.SemaphoreType.DMA,
        ],
        compiler_params=pltpu.CompilerParams(use_tc_tiling_on_sc=False),
    )
    def gather_kernel(table_hbm, idx_hbm, out_hbm, idx_v, rows_v, sem):
        wid = lax.axis_index("s") * 2 + lax.axis_index("c")
        base = wid * TOK_PER_W
        pltpu.sync_copy(idx_hbm.at[pl.ds(base, TOK_PER_W)], idx_v)
        copies = []
        for j in range(NCHUNK):
            copies.append(
                pltpu.async_copy(
                    table_hbm.at[idx_v.at[pl.ds(j * CHUNK, CHUNK)]],
                    rows_v.at[pl.ds(j * CHUNK, CHUNK)],
                    sem,
                )
            )
        for c in copies:
            c.wait()
        pltpu.sync_copy(rows_v, out_hbm.at[wid])

    return gather_kernel(table, idx)


def _fc1_body(emb_ref, w1_ref, b1_ref, h_ref):
    acc = jnp.zeros((BATCH, HIDDEN), jnp.float32)
    for s in range(NPAIR):
        e_s = emb_ref[pl.ds(s * BATCH, BATCH), :]
        w_s = w1_ref[:, pl.ds(s * PW, PW)]
        acc += lax.dot_general(
            e_s, w_s, (((1,), (1,)), ((), ())),
            preferred_element_type=jnp.float32,
        )
    h_ref[...] = jnp.maximum(acc + b1_ref[...], 0.0).astype(jnp.bfloat16)


def _fc1(emb2, W1, b1):
    return pl.pallas_call(
        _fc1_body,
        out_shape=jax.ShapeDtypeStruct((BATCH, HIDDEN), jnp.bfloat16),
    )(emb2, W1, b1.reshape(1, HIDDEN))


def _fc2_body(h_ref, w2_ref, b2_ref, out_ref):
    acc = lax.dot_general(
        w2_ref[...].astype(jnp.bfloat16), h_ref[...], (((1,), (1,)), ((), ())),
        preferred_element_type=jnp.float32,
    )
    out_ref[...] = acc + b2_ref[...].reshape(VB, 1)


def _fc2(h, W2, b2):
    nb = pl.cdiv(VOCAB, VB)
    return pl.pallas_call(
        _fc2_body,
        grid=(nb,),
        in_specs=[
            pl.BlockSpec((BATCH, HIDDEN), lambda i: (0, 0)),
            pl.BlockSpec((VB, HIDDEN), lambda i: (i, 0)),
            pl.BlockSpec((VB,), lambda i: (i,)),
        ],
        out_specs=pl.BlockSpec((VB, BATCH), lambda i: (i, 0)),
        out_shape=jax.ShapeDtypeStruct((VOCAB, BATCH), jnp.float32),
        compiler_params=pltpu.CompilerParams(
            vmem_limit_bytes=100 * 1024 * 1024,
        ),
    )(h, W2, b2)


def kernel(x, table, W1, b1, W2, b2):
    xp = x.reshape(BATCH, NPAIR, 2).transpose(1, 0, 2).reshape(NTOK)
    emb3 = _sc_gather(table, xp)
    emb2 = emb3.reshape(NPAIR * BATCH, PW)
    h = _fc1(emb2, W1, b1)
    return _fc2(h, W2, b2).T


# speedup vs baseline: 1.2351x; 1.0828x over previous
import functools

import jax
import jax.numpy as jnp
from jax import lax
from jax.experimental import pallas as pl
from jax.experimental.pallas import tpu as pltpu
from jax.experimental.pallas import tpu_sc as plsc

VOCAB, EMBED, HIDDEN, NGRAM, BATCH = 100000, 64, 512, 20, 1024
EPAD = 128
NTOK = BATCH * NGRAM
NW = 32
TOK_PER_W = NTOK // NW
CHUNK = 128
NCHUNK = TOK_PER_W // CHUNK

VT = 8192
VB = 4096


def _tpad_body(tt_ref, out_ref):
    out_ref[:, :EMBED] = tt_ref[...].T
    out_ref[:, EMBED:] = jnp.zeros((VT, EPAD - EMBED), jnp.float32)


def _transpose_pad(tableT):
    nb = pl.cdiv(VOCAB, VT)
    return pl.pallas_call(
        _tpad_body,
        grid=(nb,),
        in_specs=[pl.BlockSpec((EMBED, VT), lambda i: (0, i))],
        out_specs=pl.BlockSpec((VT, EPAD), lambda i: (i, 0)),
        out_shape=jax.ShapeDtypeStruct((VOCAB, EPAD), jnp.float32),
    )(tableT)


def _sc_gather(table_pad, idx):
    mesh = plsc.VectorSubcoreMesh(core_axis_name="c", subcore_axis_name="s")

    @functools.partial(
        pl.kernel,
        mesh=mesh,
        out_type=jax.ShapeDtypeStruct((NTOK, EPAD), jnp.float32),
        scratch_types=[
            pltpu.VMEM((TOK_PER_W,), jnp.int32),
            pltpu.VMEM((TOK_PER_W, EPAD), jnp.float32),
            pltpu.SemaphoreType.DMA,
        ],
    )
    def gather_kernel(table_hbm, idx_hbm, out_hbm, idx_v, rows_v, sem):
        wid = lax.axis_index("s") * 2 + lax.axis_index("c")
        base = wid * TOK_PER_W
        pltpu.sync_copy(idx_hbm.at[pl.ds(base, TOK_PER_W)], idx_v)
        copies = []
        for j in range(NCHUNK):
            copies.append(
                pltpu.async_copy(
                    table_hbm.at[idx_v.at[pl.ds(j * CHUNK, CHUNK)]],
                    rows_v.at[pl.ds(j * CHUNK, CHUNK)],
                    sem,
                )
            )
        for c in copies:
            c.wait()
        pltpu.sync_copy(rows_v, out_hbm.at[pl.ds(base, TOK_PER_W)])

    return gather_kernel(table_pad, idx)


def _fc1_body(emb_ref, w1_ref, b1_ref, h_ref):
    acc = jnp.zeros((BATCH, HIDDEN), jnp.float32)
    for t in range(NGRAM):
        e_t = emb_ref[pl.ds(t * BATCH, BATCH), :]
        w_t = w1_ref[:, pl.ds(t * EPAD, EPAD)]
        acc += lax.dot_general(
            e_t, w_t, (((1,), (1,)), ((), ())),
            preferred_element_type=jnp.float32,
        )
    h_ref[...] = jnp.maximum(acc + b1_ref[...], 0.0).astype(jnp.bfloat16)


def _fc1(emb, W1p, b1):
    return pl.pallas_call(
        _fc1_body,
        out_shape=jax.ShapeDtypeStruct((BATCH, HIDDEN), jnp.bfloat16),
    )(emb, W1p, b1.reshape(1, HIDDEN))


def _fc2_body(h_ref, w2_ref, b2_ref, out_ref):
    acc = lax.dot_general(
        w2_ref[...].astype(jnp.bfloat16), h_ref[...], (((1,), (1,)), ((), ())),
        preferred_element_type=jnp.float32,
    )
    out_ref[...] = acc + b2_ref[...].reshape(VB, 1)


def _fc2(h, W2, b2):
    nb = pl.cdiv(VOCAB, VB)
    return pl.pallas_call(
        _fc2_body,
        grid=(nb,),
        in_specs=[
            pl.BlockSpec((BATCH, HIDDEN), lambda i: (0, 0)),
            pl.BlockSpec((VB, HIDDEN), lambda i: (i, 0)),
            pl.BlockSpec((VB,), lambda i: (i,)),
        ],
        out_specs=pl.BlockSpec((VB, BATCH), lambda i: (i, 0)),
        out_shape=jax.ShapeDtypeStruct((VOCAB, BATCH), jnp.float32),
        compiler_params=pltpu.CompilerParams(
            vmem_limit_bytes=100 * 1024 * 1024,
        ),
    )(h, W2, b2)


def kernel(x, table, W1, b1, W2, b2):
    table_pad = _transpose_pad(table.T)
    idx = x.T.reshape(NTOK)
    emb = _sc_gather(table_pad, idx)
    W1p = jnp.pad(W1.reshape(HIDDEN, NGRAM, EMBED),
                  ((0, 0), (0, 0), (0, EPAD - EMBED))).reshape(HIDDEN, -1)
    h = _fc1(emb, W1p, b1)
    return _fc2(h, W2, b2).T
